# baseline (speedup 1.0000x reference)
import jax
import jax.numpy as jnp
from jax.experimental import pallas as pl
from jax.experimental.pallas import tpu as pltpu

_NCLS = 21
_D0, _D1, _D2 = 64, 512, 512
_CORES = 2
_STEPS = 2
_B0 = _D0 // (_CORES * _STEPS)


def _presence_body(x_ref, out_ref):
    s = pl.program_id(1)
    x = x_ref[...].reshape(_B0 * _D1, _D2)
    m = jnp.left_shift(jnp.int32(1), x)
    r = _B0 * _D1
    while r > 1:
        h = r // 2
        m = m[:h] | m[h:]
        r = h

    @pl.when(s == 0)
    def _():
        out_ref[...] = m.reshape(1, 1, _D2)

    @pl.when(s != 0)
    def _():
        out_ref[...] = out_ref[...] | m.reshape(1, 1, _D2)


def _finalize_body(p_ref, out_ref):
    m = p_ref[0] | p_ref[1]
    total = jnp.int32(0)
    for c in range(1, _NCLS):
        total = total + jnp.max((m >> c) & 1)
    out_ref[0, 0] = total.astype(jnp.float32) / _NCLS


def kernel(y_pred, y_true):
    partial = pl.pallas_call(
        _presence_body,
        grid=(_CORES, _STEPS),
        in_specs=[
            pl.BlockSpec((_B0, _D1, _D2), lambda c, s: (c * _STEPS + s, 0, 0))
        ],
        out_specs=pl.BlockSpec((1, 1, _D2), lambda c, s: (c, 0, 0)),
        out_shape=jax.ShapeDtypeStruct((_CORES, 1, _D2), jnp.int32),
        compiler_params=pltpu.CompilerParams(
            dimension_semantics=("parallel", "arbitrary"),
        ),
        name="presence_scan",
    )(y_pred)

    out = pl.pallas_call(
        _finalize_body,
        out_specs=pl.BlockSpec(memory_space=pltpu.SMEM),
        out_shape=jax.ShapeDtypeStruct((1, 1), jnp.float32),
        name="presence_finalize",
    )(partial)
    return out[0, 0]


# speedup vs baseline: 1.0562x; 1.0562x over previous
import jax
import jax.numpy as jnp
from jax.experimental import pallas as pl
from jax.experimental.pallas import tpu as pltpu

_NCLS = 21
_D0, _D1, _D2 = 64, 512, 512
_CORES = 2
_STEPS = 4
_B0 = _D0 // (_CORES * _STEPS)


def _presence_body(x_ref, out_ref):
    s = pl.program_id(1)
    x = x_ref[...].reshape(_B0 * _D1, _D2)
    m = jnp.left_shift(jnp.int32(1), x)
    r = _B0 * _D1
    while r > 1:
        h = r // 2
        m = m[:h] | m[h:]
        r = h

    @pl.when(s == 0)
    def _():
        out_ref[...] = m.reshape(1, 1, _D2)

    @pl.when(s != 0)
    def _():
        out_ref[...] = out_ref[...] | m.reshape(1, 1, _D2)


def _finalize_body(p_ref, out_ref):
    m = p_ref[0] | p_ref[1]
    total = jnp.int32(0)
    for c in range(1, _NCLS):
        total = total + jnp.max((m >> c) & 1)
    out_ref[0, 0] = total.astype(jnp.float32) / _NCLS


def kernel(y_pred, y_true):
    partial = pl.pallas_call(
        _presence_body,
        grid=(_CORES, _STEPS),
        in_specs=[
            pl.BlockSpec((_B0, _D1, _D2), lambda c, s: (c * _STEPS + s, 0, 0))
        ],
        out_specs=pl.BlockSpec((1, 1, _D2), lambda c, s: (c, 0, 0)),
        out_shape=jax.ShapeDtypeStruct((_CORES, 1, _D2), jnp.int32),
        compiler_params=pltpu.CompilerParams(
            dimension_semantics=("parallel", "arbitrary"),
        ),
        name="presence_scan",
    )(y_pred)

    out = pl.pallas_call(
        _finalize_body,
        out_specs=pl.BlockSpec(memory_space=pltpu.SMEM),
        out_shape=jax.ShapeDtypeStruct((1, 1), jnp.float32),
        name="presence_finalize",
    )(partial)
    return out[0, 0]


from kernel_1core import kernel_1core as _k1

def kernel_2core(y_pred, y_true):
    return kernel(y_pred, y_true)

kernel = _k1


# speedup vs baseline: 1.0845x; 1.0268x over previous
import jax
import jax.numpy as jnp
from jax.experimental import pallas as pl
from jax.experimental.pallas import tpu as pltpu

_NCLS = 21
_D0, _D1, _D2 = 64, 512, 512
_STEPS = 8
_B0 = _D0 // _STEPS


def _presence_body(x_ref, out_ref, acc_ref):
    s = pl.program_id(0)
    x = x_ref[...].reshape(_B0 * _D1, _D2)
    m = jnp.left_shift(jnp.int32(1), x)
    r = _B0 * _D1
    while r > 1:
        h = r // 2
        m = m[:h] | m[h:]
        r = h

    @pl.when(s == 0)
    def _():
        acc_ref[...] = m

    @pl.when(s != 0)
    def _():
        acc_ref[...] = acc_ref[...] | m

    @pl.when(s == _STEPS - 1)
    def _():
        mm = acc_ref[...]
        total = jnp.int32(0)
        for c in range(1, _NCLS):
            total = total + jnp.max((mm >> c) & 1)
        out_ref[0, 0] = total.astype(jnp.float32) / _NCLS


def kernel(y_pred, y_true):
    out = pl.pallas_call(
        _presence_body,
        grid=(_STEPS,),
        in_specs=[pl.BlockSpec((_B0, _D1, _D2), lambda s: (s, 0, 0))],
        out_specs=pl.BlockSpec(memory_space=pltpu.SMEM),
        out_shape=jax.ShapeDtypeStruct((1, 1), jnp.float32),
        scratch_shapes=[pltpu.VMEM((1, _D2), jnp.int32)],
        compiler_params=pltpu.CompilerParams(
            dimension_semantics=("arbitrary",),
        ),
        name="presence_scan",
    )(y_pred)
    return out[0, 0]
